# Initial kernel scaffold; baseline (speedup 1.0000x reference)
import jax
import jax.numpy as jnp
from jax.experimental import pallas as pl


def kernel(X, H, theta1, b1, theta2, b2, theta_out, b_out):
    raise NotImplementedError("write your pallas kernel here")


# speedup vs baseline: 2.7452x; 2.7452x over previous
import functools

import jax
import jax.numpy as jnp
from jax import lax
from jax.experimental import pallas as pl
from jax.experimental.pallas import tpu as pltpu
from jax.experimental.pallas import tpu_sc as plsc

N = 10000
M = 10000
K = 160000
FT = 256
C = 40
CP = 64

RP = 10240
NC = 2
NS = 16
CH = 128
KPAD = 163840
NCH = KPAD // (NS * CH)
DRN = RP // NS
DCH = 64
IDC = 16
BN = 1024


def _mesh():
    return plsc.VectorSubcoreMesh(core_axis_name="c", subcore_axis_name="s")


@functools.partial(
    pl.kernel,
    out_type=jax.ShapeDtypeStruct((NC, RP, 16), jnp.float32),
    mesh=_mesh(),
    compiler_params=pltpu.CompilerParams(use_tc_tiling_on_sc=False),
    scratch_types=[
        pltpu.VMEM((NCH, CH), jnp.int32),
        pltpu.VMEM((CH, 16), jnp.float32),
        pltpu.VMEM((DCH, 16), jnp.float32),
        pltpu.VMEM_SHARED((RP, 16), jnp.float32),
    ],
)
def _counts_kernel(idx_all, out, idx_v, ones_v, dbuf, acc):
    c = lax.axis_index("c")
    s = lax.axis_index("s")
    r0 = s * DRN
    one = jnp.ones((16,), jnp.float32)
    zero = jnp.zeros((16,), jnp.float32)

    @pl.loop(0, CH)
    def _(i):
        ones_v[i, :] = one

    @pl.loop(0, DCH)
    def _(i):
        dbuf[i, :] = zero

    for k in range(DRN // DCH):
        pltpu.sync_copy(dbuf, acc.at[pl.ds(r0 + k * DCH, DCH)])

    pltpu.sync_copy(idx_all.at[c, s], idx_v)
    plsc.subcore_barrier()

    @pl.loop(0, NCH)
    def _(j):
        pltpu.sync_copy(ones_v, acc.at[idx_v.at[j]], add=True)

    plsc.subcore_barrier()

    for k in range(DRN // DCH):
        pltpu.sync_copy(acc.at[pl.ds(r0 + k * DCH, DCH)], dbuf)

        @pl.loop(0, DCH)
        def _(i):
            cnt = dbuf[i, :]
            dbuf[i, :] = 1.0 / jnp.maximum(cnt, 1.0)

        pltpu.sync_copy(dbuf, out.at[c, pl.ds(r0 + k * DCH, DCH)])


def _make_seg_pass(dh, act):
    @functools.partial(
        pl.kernel,
        out_type=jax.ShapeDtypeStruct((NC, RP, dh), jnp.float32),
        mesh=_mesh(),
        compiler_params=pltpu.CompilerParams(use_tc_tiling_on_sc=(dh % 128 == 0)),
        scratch_types=[
            pltpu.VMEM((IDC, CH), jnp.int32),
            pltpu.VMEM((IDC, CH), jnp.int32),
            pltpu.VMEM((CH, dh), jnp.float32),
            pltpu.VMEM((DCH, dh), jnp.float32),
            pltpu.VMEM((DCH, 16), jnp.float32),
            pltpu.VMEM((dh,), jnp.float32),
            pltpu.VMEM_SHARED((RP, dh), jnp.float32),
        ],
    )
    def seg_pass(src, gidx, sidx, recip, bias, out,
                 gidx_v, sidx_v, rows_v, dbuf, recip_v, bias_v, acc):
        c = lax.axis_index("c")
        s = lax.axis_index("s")
        r0 = s * DRN
        zero = jnp.zeros((16,), jnp.float32)

        @pl.loop(0, DCH)
        def _(i):
            for v in range(dh // 16):
                dbuf[i, pl.ds(v * 16, 16)] = zero

        for k in range(DRN // DCH):
            pltpu.sync_copy(dbuf, acc.at[pl.ds(r0 + k * DCH, DCH)])

        pltpu.sync_copy(bias.at[c], bias_v)
        plsc.subcore_barrier()

        for b in range(NCH // IDC):
            pltpu.sync_copy(gidx.at[c, s, pl.ds(b * IDC, IDC)], gidx_v)
            pltpu.sync_copy(sidx.at[s, pl.ds(b * IDC, IDC)], sidx_v)

            @pl.loop(0, IDC)
            def _(j):
                pltpu.sync_copy(src.at[gidx_v.at[j]], rows_v)
                pltpu.sync_copy(rows_v, acc.at[sidx_v.at[j]], add=True)

        plsc.subcore_barrier()

        for k in range(DRN // DCH):
            pltpu.sync_copy(acc.at[pl.ds(r0 + k * DCH, DCH)], dbuf)
            pltpu.sync_copy(recip.at[pl.ds(r0 + k * DCH, DCH)], recip_v)

            @pl.loop(0, DCH)
            def _(i):
                rv = recip_v[i, :]
                for v in range(dh // 16):
                    x = dbuf[i, pl.ds(v * 16, 16)]
                    y = x * rv + bias_v[pl.ds(v * 16, 16)]
                    if act:
                        y = jnp.maximum(y, 0.01 * y)
                    dbuf[i, pl.ds(v * 16, 16)] = y

            pltpu.sync_copy(dbuf, out.at[c, pl.ds(r0 + k * DCH, DCH)])

    return seg_pass


_seg128 = _make_seg_pass(128, act=False)
_seg128_act = _make_seg_pass(128, act=True)
_seg32 = _make_seg_pass(32, act=False)


def _mm_std(x, w):
    kd, dd = w.shape

    def body(x_ref, w_ref, o_ref):
        o_ref[...] = jnp.dot(x_ref[...], w_ref[...],
                             preferred_element_type=jnp.float32)

    return pl.pallas_call(
        body,
        grid=(RP // BN,),
        in_specs=[pl.BlockSpec((BN, kd), lambda i: (i, 0)),
                  pl.BlockSpec((kd, dd), lambda i: (0, 0))],
        out_specs=pl.BlockSpec((BN, dd), lambda i: (i, 0)),
        out_shape=jax.ShapeDtypeStruct((RP, dd), jnp.float32),
    )(x, w)


def _mm_cm(x, w):
    kd, dd = w.shape
    kh = kd // NC

    def body(x_ref, w_ref, o_ref):
        o_ref[...] = (
            jnp.dot(x_ref[0], w_ref[:kh], preferred_element_type=jnp.float32)
            + jnp.dot(x_ref[1], w_ref[kh:], preferred_element_type=jnp.float32)
        )

    return pl.pallas_call(
        body,
        grid=(RP // BN,),
        in_specs=[pl.BlockSpec((NC, BN, kh), lambda i: (0, i, 0)),
                  pl.BlockSpec((kd, dd), lambda i: (0, 0))],
        out_specs=pl.BlockSpec((BN, dd), lambda i: (i, 0)),
        out_shape=jax.ShapeDtypeStruct((RP, dd), jnp.float32),
    )(x, w)


def kernel(X, H, theta1, b1, theta2, b2, theta_out, b_out):
    node_idx = H[0]
    edge_idx = H[1]

    pad = jnp.full((KPAD - K,), RP - 1, jnp.int32)
    nip = jnp.concatenate([node_idx, pad])
    eip = jnp.concatenate([edge_idx, pad])
    s_e = eip.reshape(NS, NCH, CH)
    s_n = nip.reshape(NS, NCH, CH)
    cvec = jnp.arange(NC, dtype=jnp.int32).reshape(NC, 1, 1, 1)
    g_n_int = 2 * nip.reshape(1, NS, NCH, CH) + cvec
    g_e_cm = eip.reshape(1, NS, NCH, CH) + cvec * RP

    recips = _counts_kernel(jnp.stack([s_e, s_n]))
    re16 = recips[0]
    rn16 = recips[1]

    zb128 = jnp.zeros((NC, 128), jnp.float32)
    xp = jnp.pad(X, ((0, RP - N), (0, 0)))

    xt = _mm_std(xp, theta1)
    ef = _seg128(xt.reshape(NC * RP, 128), g_n_int, s_e, re16, zb128)
    h1 = _seg128_act(ef.reshape(NC * RP, 128), g_e_cm, s_n, rn16,
                     b1.reshape(NC, 128))

    xt2 = _mm_cm(h1, theta2)
    ef2 = _seg128(xt2.reshape(NC * RP, 128), g_n_int, s_e, re16, zb128)
    h2 = _seg128_act(ef2.reshape(NC * RP, 128), g_e_cm, s_n, rn16,
                     b2.reshape(NC, 128))

    th_o = jnp.pad(theta_out, ((0, 0), (0, CP - C)))
    b_o = jnp.pad(b_out, (0, CP - C)).reshape(NC, CP // NC)
    xt3 = _mm_cm(h2, th_o)
    ef3 = _seg32(xt3.reshape(NC * RP, CP // NC), g_n_int, s_e, re16,
                 jnp.zeros((NC, CP // NC), jnp.float32))
    o3 = _seg32(ef3.reshape(NC * RP, CP // NC), g_e_cm, s_n, rn16, b_o)

    out = o3[:, :N].transpose(1, 0, 2).reshape(N, CP)[:, :C]
    return out


# speedup vs baseline: 3.1087x; 1.1324x over previous
import functools

import jax
import jax.numpy as jnp
from jax import lax
from jax.experimental import pallas as pl
from jax.experimental.pallas import tpu as pltpu
from jax.experimental.pallas import tpu_sc as plsc

N = 10000
M = 10000
K = 160000
FT = 256
C = 40
CP = 64

RP = 10240
NC = 2
NS = 16
CH = 128
KPAD = 163840
NCH = KPAD // (NS * CH)
DRN = RP // NS
DCH = 32
IDC = 16
BN = 1024


def _mesh():
    return plsc.VectorSubcoreMesh(core_axis_name="c", subcore_axis_name="s")


@functools.partial(
    pl.kernel,
    out_type=jax.ShapeDtypeStruct((NC, RP, 16), jnp.float32),
    mesh=_mesh(),
    compiler_params=pltpu.CompilerParams(use_tc_tiling_on_sc=False),
    scratch_types=[
        pltpu.VMEM((NCH, CH), jnp.int32),
        pltpu.VMEM((CH, 16), jnp.float32),
        pltpu.VMEM((DCH, 16), jnp.float32),
        pltpu.VMEM_SHARED((RP, 16), jnp.float32),
    ],
)
def _counts_kernel(idx_all, out, idx_v, ones_v, dbuf, acc):
    c = lax.axis_index("c")
    s = lax.axis_index("s")
    r0 = s * DRN
    one = jnp.ones((16,), jnp.float32)
    zero = jnp.zeros((16,), jnp.float32)

    @pl.loop(0, CH)
    def _(i):
        ones_v[i, :] = one

    @pl.loop(0, DCH)
    def _(i):
        dbuf[i, :] = zero

    for k in range(DRN // DCH):
        pltpu.sync_copy(dbuf, acc.at[pl.ds(r0 + k * DCH, DCH)])

    pltpu.sync_copy(idx_all.at[c, s], idx_v)
    plsc.subcore_barrier()

    @pl.loop(0, NCH)
    def _(j):
        pltpu.sync_copy(ones_v, acc.at[idx_v.at[j]], add=True)

    plsc.subcore_barrier()

    for k in range(DRN // DCH):
        pltpu.sync_copy(acc.at[pl.ds(r0 + k * DCH, DCH)], dbuf)

        @pl.loop(0, DCH)
        def _(i):
            cnt = dbuf[i, :]
            dbuf[i, :] = 1.0 / jnp.maximum(cnt, 1.0)

        pltpu.sync_copy(dbuf, out.at[c, pl.ds(r0 + k * DCH, DCH)])


def _make_seg_pass(dh, act):
    @functools.partial(
        pl.kernel,
        out_type=jax.ShapeDtypeStruct((NC, RP, dh), jnp.float32),
        mesh=_mesh(),
        compiler_params=pltpu.CompilerParams(use_tc_tiling_on_sc=(dh % 128 == 0)),
        scratch_types=[
            pltpu.VMEM((IDC, CH), jnp.int32),
            pltpu.VMEM((IDC, CH), jnp.int32),
            pltpu.VMEM((2, CH, dh), jnp.float32),
            pltpu.VMEM((DCH, dh), jnp.float32),
            pltpu.VMEM((DCH, 16), jnp.float32),
            pltpu.VMEM((dh,), jnp.float32),
            pltpu.VMEM_SHARED((RP, dh), jnp.float32),
            pltpu.SemaphoreType.DMA,
            pltpu.SemaphoreType.DMA,
        ],
    )
    def seg_pass(src, gidx, sidx, recip, bias, out,
                 gidx_v, sidx_v, rows_v, dbuf, recip_v, bias_v, acc,
                 sem0, sem1):
        c = lax.axis_index("c")
        s = lax.axis_index("s")
        r0 = s * DRN
        zero = jnp.zeros((16,), jnp.float32)

        @pl.loop(0, DCH)
        def _(i):
            for v in range(dh // 16):
                dbuf[i, pl.ds(v * 16, 16)] = zero

        for k in range(DRN // DCH):
            pltpu.sync_copy(dbuf, acc.at[pl.ds(r0 + k * DCH, DCH)])

        pltpu.sync_copy(bias.at[c], bias_v)
        plsc.subcore_barrier()

        for b in range(NCH // IDC):
            pltpu.sync_copy(gidx.at[c, s, pl.ds(b * IDC, IDC)], gidx_v)
            pltpu.sync_copy(sidx.at[s, pl.ds(b * IDC, IDC)], sidx_v)
            pltpu.async_copy(src.at[gidx_v.at[0]], rows_v.at[0], sem0)

            @pl.loop(0, IDC, step=2)
            def _(j):
                pltpu.async_copy(src.at[gidx_v.at[j + 1]], rows_v.at[1], sem1)
                pltpu.make_async_copy(src.at[gidx_v.at[j]],
                                      rows_v.at[0], sem0).wait()
                pltpu.sync_copy(rows_v.at[0], acc.at[sidx_v.at[j]], add=True)

                @pl.when(j + 2 < IDC)
                def _():
                    pltpu.async_copy(src.at[gidx_v.at[j + 2]],
                                     rows_v.at[0], sem0)

                pltpu.make_async_copy(src.at[gidx_v.at[j + 1]],
                                      rows_v.at[1], sem1).wait()
                pltpu.sync_copy(rows_v.at[1], acc.at[sidx_v.at[j + 1]],
                                add=True)

        plsc.subcore_barrier()

        for k in range(DRN // DCH):
            pltpu.sync_copy(acc.at[pl.ds(r0 + k * DCH, DCH)], dbuf)
            pltpu.sync_copy(recip.at[pl.ds(r0 + k * DCH, DCH)], recip_v)

            @pl.loop(0, DCH)
            def _(i):
                rv = recip_v[i, :]
                for v in range(dh // 16):
                    x = dbuf[i, pl.ds(v * 16, 16)]
                    y = x * rv + bias_v[pl.ds(v * 16, 16)]
                    if act:
                        y = jnp.maximum(y, 0.01 * y)
                    dbuf[i, pl.ds(v * 16, 16)] = y

            pltpu.sync_copy(dbuf, out.at[c, pl.ds(r0 + k * DCH, DCH)])

    return seg_pass


_seg128 = _make_seg_pass(128, act=False)
_seg128_act = _make_seg_pass(128, act=True)
_seg32 = _make_seg_pass(32, act=False)


def _mm_std(x, w):
    kd, dd = w.shape

    def body(x_ref, w_ref, o_ref):
        o_ref[...] = jnp.dot(x_ref[...], w_ref[...],
                             preferred_element_type=jnp.float32)

    return pl.pallas_call(
        body,
        grid=(RP // BN,),
        in_specs=[pl.BlockSpec((BN, kd), lambda i: (i, 0)),
                  pl.BlockSpec((kd, dd), lambda i: (0, 0))],
        out_specs=pl.BlockSpec((BN, dd), lambda i: (i, 0)),
        out_shape=jax.ShapeDtypeStruct((RP, dd), jnp.float32),
    )(x, w)


def _mm_cm(x, w):
    kd, dd = w.shape
    kh = kd // NC

    def body(x_ref, w_ref, o_ref):
        o_ref[...] = (
            jnp.dot(x_ref[0], w_ref[:kh], preferred_element_type=jnp.float32)
            + jnp.dot(x_ref[1], w_ref[kh:], preferred_element_type=jnp.float32)
        )

    return pl.pallas_call(
        body,
        grid=(RP // BN,),
        in_specs=[pl.BlockSpec((NC, BN, kh), lambda i: (0, i, 0)),
                  pl.BlockSpec((kd, dd), lambda i: (0, 0))],
        out_specs=pl.BlockSpec((BN, dd), lambda i: (i, 0)),
        out_shape=jax.ShapeDtypeStruct((RP, dd), jnp.float32),
    )(x, w)


def kernel(X, H, theta1, b1, theta2, b2, theta_out, b_out):
    node_idx = H[0]
    edge_idx = H[1]

    pad = jnp.full((KPAD - K,), RP - 1, jnp.int32)
    nip = jnp.concatenate([node_idx, pad])
    eip = jnp.concatenate([edge_idx, pad])
    s_e = eip.reshape(NS, NCH, CH)
    s_n = nip.reshape(NS, NCH, CH)
    cvec = jnp.arange(NC, dtype=jnp.int32).reshape(NC, 1, 1, 1)
    g_n_int = 2 * nip.reshape(1, NS, NCH, CH) + cvec
    g_e_cm = eip.reshape(1, NS, NCH, CH) + cvec * RP

    recips = _counts_kernel(jnp.stack([s_e, s_n]))
    re16 = recips[0]
    rn16 = recips[1]

    zb128 = jnp.zeros((NC, 128), jnp.float32)
    xp = jnp.pad(X, ((0, RP - N), (0, 0)))

    xt = _mm_std(xp, theta1)
    ef = _seg128(xt.reshape(NC * RP, 128), g_n_int, s_e, re16, zb128)
    h1 = _seg128_act(ef.reshape(NC * RP, 128), g_e_cm, s_n, rn16,
                     b1.reshape(NC, 128))

    xt2 = _mm_cm(h1, theta2)
    ef2 = _seg128(xt2.reshape(NC * RP, 128), g_n_int, s_e, re16, zb128)
    h2 = _seg128_act(ef2.reshape(NC * RP, 128), g_e_cm, s_n, rn16,
                     b2.reshape(NC, 128))

    th_o = jnp.pad(theta_out, ((0, 0), (0, CP - C)))
    b_o = jnp.pad(b_out, (0, CP - C)).reshape(NC, CP // NC)
    xt3 = _mm_cm(h2, th_o)
    ef3 = _seg32(xt3.reshape(NC * RP, CP // NC), g_n_int, s_e, re16,
                 jnp.zeros((NC, CP // NC), jnp.float32))
    o3 = _seg32(ef3.reshape(NC * RP, CP // NC), g_e_cm, s_n, rn16, b_o)

    out = o3[:, :N].transpose(1, 0, 2).reshape(N, CP)[:, :C]
    return out


# speedup vs baseline: 4.1331x; 1.3295x over previous
import functools

import jax
import jax.numpy as jnp
from jax import lax
from jax.experimental import pallas as pl
from jax.experimental.pallas import tpu as pltpu
from jax.experimental.pallas import tpu_sc as plsc

N = 10000
M = 10000
K = 160000
C = 40
CP = 64

RP = 10240
NC = 2
NS = 16
CH = 128
KPAD = 163840
NCH = KPAD // (NC * NS * CH)
IDC = 8
DRN = RP // NS
DCH = 64
BN = 1024


def _mesh():
    return plsc.VectorSubcoreMesh(core_axis_name="c", subcore_axis_name="s")


@functools.partial(
    pl.kernel,
    out_type=jax.ShapeDtypeStruct((NC, RP, 16), jnp.float32),
    mesh=_mesh(),
    compiler_params=pltpu.CompilerParams(use_tc_tiling_on_sc=False),
    scratch_types=[
        pltpu.VMEM((NC * NCH, CH), jnp.int32),
        pltpu.VMEM((CH, 16), jnp.float32),
        pltpu.VMEM((DCH, 16), jnp.float32),
        pltpu.VMEM_SHARED((RP, 16), jnp.float32),
    ],
)
def _counts_kernel(idx_all, out, idx_v, ones_v, dbuf, acc):
    c = lax.axis_index("c")
    s = lax.axis_index("s")
    r0 = s * DRN
    one = jnp.ones((16,), jnp.float32)
    zero = jnp.zeros((16,), jnp.float32)

    @pl.loop(0, CH)
    def _(i):
        ones_v[i, :] = one

    @pl.loop(0, DCH)
    def _(i):
        dbuf[i, :] = zero

    for k in range(DRN // DCH):
        pltpu.sync_copy(dbuf, acc.at[pl.ds(r0 + k * DCH, DCH)])

    pltpu.sync_copy(idx_all.at[c, s], idx_v)
    plsc.subcore_barrier()

    @pl.loop(0, NC * NCH)
    def _(j):
        pltpu.sync_copy(ones_v, acc.at[idx_v.at[j]], add=True)

    plsc.subcore_barrier()

    for k in range(DRN // DCH):
        pltpu.sync_copy(acc.at[pl.ds(r0 + k * DCH, DCH)], dbuf)

        @pl.loop(0, DCH)
        def _(i):
            cnt = dbuf[i, :]
            dbuf[i, :] = 1.0 / jnp.maximum(cnt, 1.0)

        pltpu.sync_copy(dbuf, out.at[c, pl.ds(r0 + k * DCH, DCH)])


def _make_seg_pass(dh):
    @functools.partial(
        pl.kernel,
        out_type=jax.ShapeDtypeStruct((NC, RP, dh), jnp.bfloat16),
        mesh=_mesh(),
        compiler_params=pltpu.CompilerParams(use_tc_tiling_on_sc=False),
        scratch_types=[
            pltpu.VMEM((IDC, CH), jnp.int32),
            pltpu.VMEM((IDC, CH), jnp.int32),
            pltpu.VMEM((2, CH, dh), jnp.bfloat16),
            pltpu.VMEM((DCH, dh), jnp.bfloat16),
            pltpu.VMEM_SHARED((RP, dh), jnp.bfloat16),
            pltpu.SemaphoreType.DMA,
            pltpu.SemaphoreType.DMA,
        ],
    )
    def seg_pass(src, gidx, sidx, out,
                 gidx_v, sidx_v, rows_v, zbuf, acc, sem0, sem1):
        c = lax.axis_index("c")
        s = lax.axis_index("s")
        r0 = s * DRN
        zero = jnp.zeros((32,), jnp.bfloat16)

        @pl.loop(0, DCH)
        def _(i):
            for v in range(dh // 32):
                zbuf[i, pl.ds(v * 32, 32)] = zero

        for k in range(DRN // DCH):
            pltpu.sync_copy(zbuf, acc.at[pl.ds(r0 + k * DCH, DCH)])

        plsc.subcore_barrier()

        for b in range(NCH // IDC):
            pltpu.sync_copy(gidx.at[c, s, pl.ds(b * IDC, IDC)], gidx_v)
            pltpu.sync_copy(sidx.at[c, s, pl.ds(b * IDC, IDC)], sidx_v)
            pltpu.async_copy(src.at[gidx_v.at[0]], rows_v.at[0], sem0)

            @pl.loop(0, IDC, step=2)
            def _(j):
                pltpu.async_copy(src.at[gidx_v.at[j + 1]], rows_v.at[1], sem1)
                pltpu.make_async_copy(src.at[gidx_v.at[j]],
                                      rows_v.at[0], sem0).wait()
                pltpu.sync_copy(rows_v.at[0], acc.at[sidx_v.at[j]], add=True)

                @pl.when(j + 2 < IDC)
                def _():
                    pltpu.async_copy(src.at[gidx_v.at[j + 2]],
                                     rows_v.at[0], sem0)

                pltpu.make_async_copy(src.at[gidx_v.at[j + 1]],
                                      rows_v.at[1], sem1).wait()
                pltpu.sync_copy(rows_v.at[1], acc.at[sidx_v.at[j + 1]],
                                add=True)

        plsc.subcore_barrier()

        pltpu.sync_copy(acc.at[pl.ds(r0, DRN)], out.at[c, pl.ds(r0, DRN)])

    return seg_pass


_seg256 = _make_seg_pass(256)
_seg64 = _make_seg_pass(64)


def _combine(p, recip, bias, act, out_dtype):
    dd = p.shape[-1]

    def body(p_ref, r_ref, b_ref, o_ref):
        y = (p_ref[0].astype(jnp.float32) + p_ref[1].astype(jnp.float32))
        y = y * r_ref[:, 0:1] + b_ref[...]
        if act:
            y = jnp.maximum(y, 0.01 * y)
        o_ref[...] = y.astype(out_dtype)

    return pl.pallas_call(
        body,
        grid=(RP // BN,),
        in_specs=[pl.BlockSpec((NC, BN, dd), lambda i: (0, i, 0)),
                  pl.BlockSpec((BN, 16), lambda i: (i, 0)),
                  pl.BlockSpec((1, dd), lambda i: (0, 0))],
        out_specs=pl.BlockSpec((BN, dd), lambda i: (i, 0)),
        out_shape=jax.ShapeDtypeStruct((RP, dd), out_dtype),
    )(p, recip, bias.reshape(1, dd))


def _mm(x, w):
    kd, dd = w.shape

    def body(x_ref, w_ref, o_ref):
        o_ref[...] = jnp.dot(x_ref[...], w_ref[...],
                             preferred_element_type=jnp.float32
                             ).astype(jnp.bfloat16)

    return pl.pallas_call(
        body,
        grid=(RP // BN,),
        in_specs=[pl.BlockSpec((BN, kd), lambda i: (i, 0)),
                  pl.BlockSpec((kd, dd), lambda i: (0, 0))],
        out_specs=pl.BlockSpec((BN, dd), lambda i: (i, 0)),
        out_shape=jax.ShapeDtypeStruct((RP, dd), jnp.bfloat16),
    )(x, w)


def kernel(X, H, theta1, b1, theta2, b2, theta_out, b_out):
    node_idx = H[0]
    edge_idx = H[1]

    pad = jnp.full((KPAD - K,), RP - 1, jnp.int32)
    nip = jnp.concatenate([node_idx, pad])
    eip = jnp.concatenate([edge_idx, pad])
    n4 = nip.reshape(NC, NS, NCH, CH)
    e4 = eip.reshape(NC, NS, NCH, CH)
    idx_cnt = jnp.stack([eip.reshape(NS, NC * NCH, CH),
                         nip.reshape(NS, NC * NCH, CH)])

    recips = _counts_kernel(idx_cnt)
    re16 = recips[0]
    rn16 = recips[1]

    xp = jnp.pad(X, ((0, RP - N), (0, 0))).astype(jnp.bfloat16)
    zb256 = jnp.zeros((256,), jnp.float32)

    xt = _mm(xp, theta1)
    ef = _combine(_seg256(xt, n4, e4), re16, zb256, False, jnp.bfloat16)
    h1 = _combine(_seg256(ef, e4, n4), rn16, b1, True, jnp.bfloat16)

    xt2 = _mm(h1, theta2)
    ef2 = _combine(_seg256(xt2, n4, e4), re16, zb256, False, jnp.bfloat16)
    h2 = _combine(_seg256(ef2, e4, n4), rn16, b2, True, jnp.bfloat16)

    th_o = jnp.pad(theta_out, ((0, 0), (0, CP - C)))
    b_o = jnp.pad(b_out, (0, CP - C))
    xt3 = _mm(h2, th_o)
    ef3 = _combine(_seg64(xt3, n4, e4), re16, jnp.zeros((CP,), jnp.float32),
                   False, jnp.bfloat16)
    o3 = _combine(_seg64(ef3, e4, n4), rn16, b_o, False, jnp.float32)

    return o3[:N, :C]


# speedup vs baseline: 4.1449x; 1.0029x over previous
import functools

import jax
import jax.numpy as jnp
from jax import lax
from jax.experimental import pallas as pl
from jax.experimental.pallas import tpu as pltpu
from jax.experimental.pallas import tpu_sc as plsc

N = 10000
M = 10000
K = 160000
C = 40
CP = 64

RP = 10240
NC = 2
NS = 16
CH = 128
KPAD = 163840
NCH = KPAD // (NC * NS * CH)
NBUF = 2
IDC = 20
DRN = RP // NS
DCH = 32
BN = 1024


def _mesh():
    return plsc.VectorSubcoreMesh(core_axis_name="c", subcore_axis_name="s")


@functools.partial(
    pl.kernel,
    out_type=jax.ShapeDtypeStruct((NC, RP, 16), jnp.float32),
    mesh=_mesh(),
    compiler_params=pltpu.CompilerParams(use_tc_tiling_on_sc=False),
    scratch_types=[
        pltpu.VMEM((NC * NCH, CH), jnp.int32),
        pltpu.VMEM((CH, 16), jnp.float32),
        pltpu.VMEM((DCH, 16), jnp.float32),
        pltpu.VMEM_SHARED((RP, 16), jnp.float32),
        [pltpu.SemaphoreType.DMA] * 4,
    ],
)
def _counts_kernel(idx_all, out, idx_v, ones_v, dbuf, acc, csems):
    c = lax.axis_index("c")
    s = lax.axis_index("s")
    r0 = s * DRN
    one = jnp.ones((16,), jnp.float32)
    zero = jnp.zeros((16,), jnp.float32)

    @pl.loop(0, CH)
    def _(i):
        ones_v[i, :] = one

    @pl.loop(0, DCH)
    def _(i):
        dbuf[i, :] = zero

    for k in range(DRN // DCH):
        pltpu.sync_copy(dbuf, acc.at[pl.ds(r0 + k * DCH, DCH)])

    pltpu.sync_copy(idx_all.at[c, s], idx_v)
    plsc.subcore_barrier()

    @pl.loop(0, NC * NCH, step=4)
    def _(j):
        for q in range(4):
            pltpu.async_copy(ones_v, acc.at[idx_v.at[j + q]], csems[q],
                             add=True)
        for q in range(4):
            pltpu.make_async_copy(ones_v, acc.at[pl.ds(0, CH)],
                                  csems[q]).wait()

    plsc.subcore_barrier()

    for k in range(DRN // DCH):
        pltpu.sync_copy(acc.at[pl.ds(r0 + k * DCH, DCH)], dbuf)

        @pl.loop(0, DCH)
        def _(i):
            cnt = dbuf[i, :]
            dbuf[i, :] = 1.0 / jnp.maximum(cnt, 1.0)

        pltpu.sync_copy(dbuf, out.at[c, pl.ds(r0 + k * DCH, DCH)])


def _make_seg_pass(dh):
    @functools.partial(
        pl.kernel,
        out_type=jax.ShapeDtypeStruct((NC, RP, dh), jnp.bfloat16),
        mesh=_mesh(),
        compiler_params=pltpu.CompilerParams(use_tc_tiling_on_sc=False),
        scratch_types=[
            pltpu.VMEM((IDC, CH), jnp.int32),
            pltpu.VMEM((IDC, CH), jnp.int32),
            pltpu.VMEM((NBUF, CH, dh), jnp.bfloat16),
            pltpu.VMEM((DCH, dh), jnp.bfloat16),
            pltpu.VMEM_SHARED((RP, dh), jnp.bfloat16),
            [pltpu.SemaphoreType.DMA] * NBUF,
            [pltpu.SemaphoreType.DMA] * NBUF,
        ],
    )
    def seg_pass(src, gidx, sidx, out,
                 gidx_v, sidx_v, rows_v, zbuf, acc, gsems, ssems):
        c = lax.axis_index("c")
        s = lax.axis_index("s")
        r0 = s * DRN
        zero = jnp.zeros((32,), jnp.bfloat16)

        @pl.loop(0, DCH)
        def _(i):
            for v in range(dh // 32):
                zbuf[i, pl.ds(v * 32, 32)] = zero

        for k in range(DRN // DCH):
            pltpu.sync_copy(zbuf, acc.at[pl.ds(r0 + k * DCH, DCH)])

        plsc.subcore_barrier()

        for b in range(NCH // IDC):
            pltpu.sync_copy(gidx.at[c, s, pl.ds(b * IDC, IDC)], gidx_v)
            pltpu.sync_copy(sidx.at[c, s, pl.ds(b * IDC, IDC)], sidx_v)
            for q in range(NBUF):
                pltpu.async_copy(src.at[gidx_v.at[q]], rows_v.at[q],
                                 gsems[q])

            @pl.loop(0, IDC, step=NBUF)
            def _(j):
                for q in range(NBUF):
                    pltpu.make_async_copy(src.at[gidx_v.at[0]],
                                          rows_v.at[q], gsems[q]).wait()
                    pltpu.async_copy(rows_v.at[q], acc.at[sidx_v.at[j + q]],
                                     ssems[q], add=True)
                for q in range(NBUF):
                    pltpu.make_async_copy(rows_v.at[q],
                                          acc.at[pl.ds(0, CH)],
                                          ssems[q]).wait()

                    @pl.when(j + NBUF + q < IDC)
                    def _():
                        pltpu.async_copy(src.at[gidx_v.at[j + NBUF + q]],
                                         rows_v.at[q], gsems[q])

        plsc.subcore_barrier()

        pltpu.sync_copy(acc.at[pl.ds(r0, DRN)], out.at[c, pl.ds(r0, DRN)])

    return seg_pass


_seg256 = _make_seg_pass(256)
_seg64 = _make_seg_pass(64)


def _combine(p, recip, bias, act, out_dtype):
    dd = p.shape[-1]

    def body(p_ref, r_ref, b_ref, o_ref):
        y = (p_ref[0].astype(jnp.float32) + p_ref[1].astype(jnp.float32))
        y = y * r_ref[:, 0:1] + b_ref[...]
        if act:
            y = jnp.maximum(y, 0.01 * y)
        o_ref[...] = y.astype(out_dtype)

    return pl.pallas_call(
        body,
        grid=(RP // BN,),
        in_specs=[pl.BlockSpec((NC, BN, dd), lambda i: (0, i, 0)),
                  pl.BlockSpec((BN, 16), lambda i: (i, 0)),
                  pl.BlockSpec((1, dd), lambda i: (0, 0))],
        out_specs=pl.BlockSpec((BN, dd), lambda i: (i, 0)),
        out_shape=jax.ShapeDtypeStruct((RP, dd), out_dtype),
    )(p, recip, bias.reshape(1, dd))


def _mm_combine(p, recip, bias, w):
    kd, dd = w.shape

    def body(p_ref, r_ref, b_ref, w_ref, o_ref):
        x = p_ref[0].astype(jnp.float32) + p_ref[1].astype(jnp.float32)
        x = x * r_ref[:, 0:1] + b_ref[...]
        x = jnp.maximum(x, 0.01 * x)
        o_ref[...] = jnp.dot(x.astype(jnp.bfloat16), w_ref[...],
                             preferred_element_type=jnp.float32
                             ).astype(jnp.bfloat16)

    return pl.pallas_call(
        body,
        grid=(RP // BN,),
        in_specs=[pl.BlockSpec((NC, BN, kd), lambda i: (0, i, 0)),
                  pl.BlockSpec((BN, 16), lambda i: (i, 0)),
                  pl.BlockSpec((1, kd), lambda i: (0, 0)),
                  pl.BlockSpec((kd, dd), lambda i: (0, 0))],
        out_specs=pl.BlockSpec((BN, dd), lambda i: (i, 0)),
        out_shape=jax.ShapeDtypeStruct((RP, dd), jnp.bfloat16),
    )(p, recip, bias.reshape(1, kd), w)


def _mm(x, w):
    kd, dd = w.shape

    def body(x_ref, w_ref, o_ref):
        o_ref[...] = jnp.dot(x_ref[...], w_ref[...],
                             preferred_element_type=jnp.float32
                             ).astype(jnp.bfloat16)

    return pl.pallas_call(
        body,
        grid=(RP // BN,),
        in_specs=[pl.BlockSpec((BN, kd), lambda i: (i, 0)),
                  pl.BlockSpec((kd, dd), lambda i: (0, 0))],
        out_specs=pl.BlockSpec((BN, dd), lambda i: (i, 0)),
        out_shape=jax.ShapeDtypeStruct((RP, dd), jnp.bfloat16),
    )(x, w)


def kernel(X, H, theta1, b1, theta2, b2, theta_out, b_out):
    node_idx = H[0]
    edge_idx = H[1]

    pad = jnp.full((KPAD - K,), RP - 1, jnp.int32)
    nip = jnp.concatenate([node_idx, pad])
    eip = jnp.concatenate([edge_idx, pad])
    n4 = nip.reshape(NC, NS, NCH, CH)
    e4 = eip.reshape(NC, NS, NCH, CH)
    idx_cnt = jnp.stack([eip.reshape(NS, NC * NCH, CH),
                         nip.reshape(NS, NC * NCH, CH)])

    recips = _counts_kernel(idx_cnt)
    re16 = recips[0]
    rn16 = recips[1]

    xp = jnp.pad(X, ((0, RP - N), (0, 0))).astype(jnp.bfloat16)
    zb256 = jnp.zeros((256,), jnp.float32)

    xt = _mm(xp, theta1)
    ef = _combine(_seg256(xt, n4, e4), re16, zb256, False, jnp.bfloat16)
    pn1 = _seg256(ef, e4, n4)

    xt2 = _mm_combine(pn1, rn16, b1, theta2)
    ef2 = _combine(_seg256(xt2, n4, e4), re16, zb256, False, jnp.bfloat16)
    pn2 = _seg256(ef2, e4, n4)

    th_o = jnp.pad(theta_out, ((0, 0), (0, CP - C)))
    b_o = jnp.pad(b_out, (0, CP - C))
    xt3 = _mm_combine(pn2, rn16, b2, th_o)
    ef3 = _combine(_seg64(xt3, n4, e4), re16, jnp.zeros((CP,), jnp.float32),
                   False, jnp.bfloat16)
    o3 = _combine(_seg64(ef3, e4, n4), rn16, b_o, False, jnp.float32)

    return o3[:N, :C]
